# baseline (speedup 1.0000x reference)
import functools

import numpy as np
import jax
import jax.numpy as jnp
from jax import lax
from jax.experimental import pallas as pl

EMB = 256
HID = 256
NREL = 6
N = 512

_NEG = -9e15

_VINV = np.linalg.inv(np.vander(np.arange(float(NREL)), increasing=True)).astype(np.float32)


def _net_kernel(feat_ref, adj_ref, wfc1_ref, bfc1_ref,
                w0_ref, a0_ref, c0_ref,
                w1_ref, a1_ref, c1_ref,
                out_ref):
    feat = feat_ref[0]
    adj = adj_ref[0]
    adj_f = adj.astype(jnp.float32)
    mask = adj > 0
    has_nbr = jnp.any(mask, axis=1, keepdims=True)

    H = jnp.dot(feat, wfc1_ref[...], preferred_element_type=jnp.float32)
    H = jax.nn.relu(H + bfc1_ref[...])

    for w_ref, a_ref, c_ref in ((w0_ref, a0_ref, c0_ref),
                                (w1_ref, a1_ref, c1_ref)):
        Wh = jnp.dot(H, w_ref[...], preferred_element_type=jnp.float32)
        s_src = jnp.dot(Wh, a_ref[:, 0:1],
                        preferred_element_type=jnp.float32)
        s_dst = lax.dot_general(a_ref[:, 1:2], Wh,
                                (((0,), (1,)), ((), ())),
                                preferred_element_type=jnp.float32)

        rel = c_ref[0, NREL - 1]
        for k in range(NREL - 2, -1, -1):
            rel = rel * adj_f + c_ref[0, k]

        e = (s_src + s_dst) + rel
        e = jnp.maximum(e, 0.2 * e)
        e = jnp.where(mask, e, _NEG)
        m = jnp.max(e, axis=1, keepdims=True)
        p = jnp.exp(e - m)
        s = jnp.sum(p, axis=1, keepdims=True)
        inv = jnp.where(has_nbr, 1.0 / s, 0.0)
        attn = p * inv

        out = jnp.dot(attn, Wh, preferred_element_type=jnp.float32)
        out = jnp.where(out > 0, out, jnp.exp(out) - 1.0)
        H = out + H

    out_ref[0, :, :HID] = H
    out_ref[0, :, HID:] = feat


@jax.jit
def kernel(utterance_features, semantic_adj, q_type, pos,
           W_fc1, b_fc1,
           W_gat0, a_src0, a_dst0, rel_bias0,
           W_gat1, a_src1, a_dst1, rel_bias1):
    del q_type, pos
    B = utterance_features.shape[0]

    vinv = jnp.asarray(_VINV)
    coef = lambda rb: jnp.dot(vinv, rb).reshape(1, NREL)
    avec = lambda a_s, a_d: jnp.stack([a_s, a_d], axis=1)

    grid_spec = pl.GridSpec(
        grid=(B,),
        in_specs=[
            pl.BlockSpec((1, N, EMB), lambda b: (b, 0, 0)),
            pl.BlockSpec((1, N, N), lambda b: (b, 0, 0)),
            pl.BlockSpec((EMB, HID), lambda b: (0, 0)),
            pl.BlockSpec((1, HID), lambda b: (0, 0)),
            pl.BlockSpec((HID, HID), lambda b: (0, 0)),
            pl.BlockSpec((HID, 2), lambda b: (0, 0)),
            pl.BlockSpec((1, NREL), lambda b: (0, 0)),
            pl.BlockSpec((HID, HID), lambda b: (0, 0)),
            pl.BlockSpec((HID, 2), lambda b: (0, 0)),
            pl.BlockSpec((1, NREL), lambda b: (0, 0)),
        ],
        out_specs=pl.BlockSpec((1, N, HID + EMB), lambda b: (b, 0, 0)),
    )

    return pl.pallas_call(
        _net_kernel,
        grid_spec=grid_spec,
        out_shape=jax.ShapeDtypeStruct((B, N, HID + EMB), jnp.float32),
    )(utterance_features, semantic_adj,
      W_fc1, b_fc1.reshape(1, HID),
      W_gat0, avec(a_src0, a_dst0), coef(rel_bias0),
      W_gat1, avec(a_src1, a_dst1), coef(rel_bias1))


# speedup vs baseline: 1.1021x; 1.1021x over previous
import jax
import jax.numpy as jnp
from jax import lax
from jax.experimental import pallas as pl

EMB = 256
HID = 256
NREL = 6
N = 512

_NEG = -9e15


def _net_kernel(feat_ref, adj_ref, wfc1_ref, bfc1_ref,
                w0_ref, a0_ref, c0_ref,
                w1_ref, a1_ref, c1_ref,
                out_ref):
    feat = feat_ref[0]
    adj = adj_ref[0]
    mask = adj > 0
    has_nbr = jnp.any(mask, axis=1, keepdims=True)

    H = jnp.dot(feat, wfc1_ref[...], preferred_element_type=jnp.float32)
    H = jax.nn.relu(H + bfc1_ref[...])

    for w_ref, a_ref, c_ref in ((w0_ref, a0_ref, c0_ref),
                                (w1_ref, a1_ref, c1_ref)):
        Wh = jnp.dot(H, w_ref[...], preferred_element_type=jnp.float32)
        s_src = jnp.dot(Wh, a_ref[:, 0:1],
                        preferred_element_type=jnp.float32)
        s_dst = lax.dot_general(a_ref[:, 1:2], Wh,
                                (((0,), (1,)), ((), ())),
                                preferred_element_type=jnp.float32)

        rel = jnp.full((N, N), c_ref[0, 0], dtype=jnp.float32)
        for r in range(1, NREL):
            rel = jnp.where(adj == r, c_ref[0, r], rel)

        e = (s_src + s_dst) + rel
        e = jnp.maximum(e, 0.2 * e)
        e = jnp.where(mask, e, _NEG)
        m = jnp.max(e, axis=1, keepdims=True)
        p = jnp.exp(e - m)
        s = jnp.sum(p, axis=1, keepdims=True)
        inv = jnp.where(has_nbr, 1.0 / s, 0.0)
        attn = p * inv

        out = jnp.dot(attn, Wh, preferred_element_type=jnp.float32)
        out = jnp.where(out > 0, out, jnp.exp(out) - 1.0)
        H = out + H

    out_ref[0, :, :HID] = H
    out_ref[0, :, HID:] = feat


@jax.jit
def kernel(utterance_features, semantic_adj, q_type, pos,
           W_fc1, b_fc1,
           W_gat0, a_src0, a_dst0, rel_bias0,
           W_gat1, a_src1, a_dst1, rel_bias1):
    del q_type, pos
    B = utterance_features.shape[0]

    coef = lambda rb: rb.reshape(1, NREL)
    avec = lambda a_s, a_d: jnp.stack([a_s, a_d], axis=1)

    grid_spec = pl.GridSpec(
        grid=(B,),
        in_specs=[
            pl.BlockSpec((1, N, EMB), lambda b: (b, 0, 0)),
            pl.BlockSpec((1, N, N), lambda b: (b, 0, 0)),
            pl.BlockSpec((EMB, HID), lambda b: (0, 0)),
            pl.BlockSpec((1, HID), lambda b: (0, 0)),
            pl.BlockSpec((HID, HID), lambda b: (0, 0)),
            pl.BlockSpec((HID, 2), lambda b: (0, 0)),
            pl.BlockSpec((1, NREL), lambda b: (0, 0)),
            pl.BlockSpec((HID, HID), lambda b: (0, 0)),
            pl.BlockSpec((HID, 2), lambda b: (0, 0)),
            pl.BlockSpec((1, NREL), lambda b: (0, 0)),
        ],
        out_specs=pl.BlockSpec((1, N, HID + EMB), lambda b: (b, 0, 0)),
    )

    return pl.pallas_call(
        _net_kernel,
        grid_spec=grid_spec,
        out_shape=jax.ShapeDtypeStruct((B, N, HID + EMB), jnp.float32),
    )(utterance_features, semantic_adj,
      W_fc1, b_fc1.reshape(1, HID),
      W_gat0, avec(a_src0, a_dst0), coef(rel_bias0),
      W_gat1, avec(a_src1, a_dst1), coef(rel_bias1))


# speedup vs baseline: 1.1291x; 1.0245x over previous
import jax
import jax.numpy as jnp
from jax import lax
from jax.experimental import pallas as pl

EMB = 256
HID = 256
NREL = 6
N = 512

_NEG = -9e15


def _net_kernel(feat_ref, adj_ref, wfc1_ref, bfc1_ref,
                w0_ref, a0_ref, c0_ref,
                w1_ref, a1_ref, c1_ref,
                out_ref):
    feat = feat_ref[0]
    adj = adj_ref[0]
    mask = adj > 0
    has_nbr = jnp.any(mask, axis=1, keepdims=True)

    H = jnp.dot(feat, wfc1_ref[...], preferred_element_type=jnp.float32)
    H = jax.nn.relu(H + bfc1_ref[...])

    for w_ref, a_ref, c_ref in ((w0_ref, a0_ref, c0_ref),
                                (w1_ref, a1_ref, c1_ref)):
        Wh = jnp.dot(H, w_ref[...], preferred_element_type=jnp.float32)
        s_src = jnp.sum(Wh * a_ref[0:1, :], axis=1, keepdims=True)
        s_dst = jnp.sum(Wh * a_ref[1:2, :], axis=1, keepdims=True)

        rel = jnp.full((N, N), c_ref[0, 0], dtype=jnp.float32)
        for r in range(1, NREL):
            rel = jnp.where(adj == r, c_ref[0, r], rel)

        e = (s_src + s_dst.reshape(1, N)) + rel
        e = jnp.maximum(e, 0.2 * e)
        e = jnp.where(mask, e, _NEG)
        m = jnp.max(e, axis=1, keepdims=True)
        p = jnp.exp(e - m)
        s = jnp.sum(p, axis=1, keepdims=True)
        inv = jnp.where(has_nbr, 1.0 / s, 0.0)
        attn = p * inv

        out = jnp.dot(attn, Wh, preferred_element_type=jnp.float32)
        out = jnp.where(out > 0, out, jnp.exp(out) - 1.0)
        H = out + H

    out_ref[0, :, :HID] = H
    out_ref[0, :, HID:] = feat


@jax.jit
def kernel(utterance_features, semantic_adj, q_type, pos,
           W_fc1, b_fc1,
           W_gat0, a_src0, a_dst0, rel_bias0,
           W_gat1, a_src1, a_dst1, rel_bias1):
    del q_type, pos
    B = utterance_features.shape[0]

    coef = lambda rb: rb.reshape(1, NREL)
    avec = lambda a_s, a_d: jnp.stack([a_s, a_d], axis=0)

    grid_spec = pl.GridSpec(
        grid=(B,),
        in_specs=[
            pl.BlockSpec((1, N, EMB), lambda b: (b, 0, 0)),
            pl.BlockSpec((1, N, N), lambda b: (b, 0, 0)),
            pl.BlockSpec((EMB, HID), lambda b: (0, 0)),
            pl.BlockSpec((1, HID), lambda b: (0, 0)),
            pl.BlockSpec((HID, HID), lambda b: (0, 0)),
            pl.BlockSpec((2, HID), lambda b: (0, 0)),
            pl.BlockSpec((1, NREL), lambda b: (0, 0)),
            pl.BlockSpec((HID, HID), lambda b: (0, 0)),
            pl.BlockSpec((2, HID), lambda b: (0, 0)),
            pl.BlockSpec((1, NREL), lambda b: (0, 0)),
        ],
        out_specs=pl.BlockSpec((1, N, HID + EMB), lambda b: (b, 0, 0)),
    )

    return pl.pallas_call(
        _net_kernel,
        grid_spec=grid_spec,
        out_shape=jax.ShapeDtypeStruct((B, N, HID + EMB), jnp.float32),
    )(utterance_features, semantic_adj,
      W_fc1, b_fc1.reshape(1, HID),
      W_gat0, avec(a_src0, a_dst0), coef(rel_bias0),
      W_gat1, avec(a_src1, a_dst1), coef(rel_bias1))


# speedup vs baseline: 1.1919x; 1.0557x over previous
import jax
import jax.numpy as jnp
from jax import lax
from jax.experimental import pallas as pl

EMB = 256
HID = 256
NREL = 6
N = 512

_NEG = -9e15


def _net_kernel(feat_ref, adj_ref, wfc1_ref, bfc1_ref,
                w0_ref, as0_ref, ad0_ref, c0_ref,
                w1_ref, as1_ref, ad1_ref, c1_ref,
                out_ref):
    feat = feat_ref[0]
    adj = adj_ref[0]
    mask = adj > 0
    has_nbr = jnp.any(mask, axis=1, keepdims=True)

    H = jnp.dot(feat, wfc1_ref[...], preferred_element_type=jnp.float32)
    H = jax.nn.relu(H + bfc1_ref[...])

    for w_ref, as_ref, ad_ref, c_ref in (
            (w0_ref, as0_ref, ad0_ref, c0_ref),
            (w1_ref, as1_ref, ad1_ref, c1_ref)):
        Wh = jnp.dot(H, w_ref[...], preferred_element_type=jnp.float32)
        s_src = jnp.sum(Wh * as_ref[...], axis=1, keepdims=True)
        s_dst = jnp.sum(Wh * ad_ref[...], axis=1, keepdims=True)

        rel = jnp.full((N, N), c_ref[0, 0], dtype=jnp.float32)
        for r in range(1, NREL):
            rel = jnp.where(adj == r, c_ref[0, r], rel)

        e = (s_src + s_dst.reshape(1, N)) + rel
        e = jnp.maximum(e, 0.2 * e)
        e = jnp.where(mask, e, _NEG)
        m = jnp.max(e, axis=1, keepdims=True)
        p = jnp.exp(e - m)
        s = jnp.sum(p, axis=1, keepdims=True)
        inv = jnp.where(has_nbr, 1.0 / s, 0.0)
        attn = p * inv

        out = jnp.dot(attn, Wh, preferred_element_type=jnp.float32)
        out = jnp.where(out > 0, out, jnp.exp(out) - 1.0)
        H = out + H

    out_ref[0, :, :HID] = H
    out_ref[0, :, HID:] = feat


@jax.jit
def kernel(utterance_features, semantic_adj, q_type, pos,
           W_fc1, b_fc1,
           W_gat0, a_src0, a_dst0, rel_bias0,
           W_gat1, a_src1, a_dst1, rel_bias1):
    del q_type, pos
    B = utterance_features.shape[0]

    row = lambda v: v.reshape(1, -1)

    grid_spec = pl.GridSpec(
        grid=(B,),
        in_specs=[
            pl.BlockSpec((1, N, EMB), lambda b: (b, 0, 0)),
            pl.BlockSpec((1, N, N), lambda b: (b, 0, 0)),
            pl.BlockSpec((EMB, HID), lambda b: (0, 0)),
            pl.BlockSpec((1, HID), lambda b: (0, 0)),
            pl.BlockSpec((HID, HID), lambda b: (0, 0)),
            pl.BlockSpec((1, HID), lambda b: (0, 0)),
            pl.BlockSpec((1, HID), lambda b: (0, 0)),
            pl.BlockSpec((1, NREL), lambda b: (0, 0)),
            pl.BlockSpec((HID, HID), lambda b: (0, 0)),
            pl.BlockSpec((1, HID), lambda b: (0, 0)),
            pl.BlockSpec((1, HID), lambda b: (0, 0)),
            pl.BlockSpec((1, NREL), lambda b: (0, 0)),
        ],
        out_specs=pl.BlockSpec((1, N, HID + EMB), lambda b: (b, 0, 0)),
    )

    return pl.pallas_call(
        _net_kernel,
        grid_spec=grid_spec,
        out_shape=jax.ShapeDtypeStruct((B, N, HID + EMB), jnp.float32),
    )(utterance_features, semantic_adj,
      W_fc1, row(b_fc1),
      W_gat0, row(a_src0), row(a_dst0), row(rel_bias0),
      W_gat1, row(a_src1), row(a_dst1), row(rel_bias1))


# speedup vs baseline: 1.2880x; 1.0806x over previous
import jax
import jax.numpy as jnp
from jax import lax
from jax.experimental import pallas as pl

EMB = 256
HID = 256
NREL = 6
N = 512

_NEG = -9e15


def _net_kernel(feat_ref, adj_ref, wfc1_ref, bfc1_ref,
                w0_ref, as0_ref, ad0_ref, c0_ref,
                w1_ref, as1_ref, ad1_ref, c1_ref,
                out_ref):
    feat = feat_ref[0]
    adj = adj_ref[0]
    mask = adj > 0
    has_nbr = jnp.any(mask, axis=1, keepdims=True)
    adj_bf = adj.astype(jnp.bfloat16)

    H = jnp.dot(feat, wfc1_ref[...], preferred_element_type=jnp.float32)
    H = jax.nn.relu(H + bfc1_ref[...])

    for w_ref, as_ref, ad_ref, c_ref in (
            (w0_ref, as0_ref, ad0_ref, c0_ref),
            (w1_ref, as1_ref, ad1_ref, c1_ref)):
        Wh = jnp.dot(H, w_ref[...], preferred_element_type=jnp.float32)
        s_src = jnp.sum(Wh * as_ref[...], axis=1, keepdims=True)
        s_dst = jnp.sum(Wh * ad_ref[...], axis=1, keepdims=True)

        rel = jnp.full((N, N), c_ref[0, 1].astype(jnp.bfloat16),
                       dtype=jnp.bfloat16)
        for r in range(2, NREL):
            rel = jnp.where(adj_bf == r,
                            c_ref[0, r].astype(jnp.bfloat16), rel)

        e = (s_src + s_dst.reshape(1, N)) + rel.astype(jnp.float32)
        e = jnp.maximum(e, 0.2 * e)
        e = jnp.where(mask, e, _NEG)
        m = jnp.max(e, axis=1, keepdims=True)
        p = jnp.exp(e - m)
        s = jnp.sum(p, axis=1, keepdims=True)
        inv = jnp.where(has_nbr, 1.0 / s, 0.0)
        attn = p * inv

        out = jnp.dot(attn, Wh, preferred_element_type=jnp.float32)
        out = jnp.where(out > 0, out, jnp.exp(out) - 1.0)
        H = out + H

    out_ref[0, :, :HID] = H
    out_ref[0, :, HID:] = feat


@jax.jit
def kernel(utterance_features, semantic_adj, q_type, pos,
           W_fc1, b_fc1,
           W_gat0, a_src0, a_dst0, rel_bias0,
           W_gat1, a_src1, a_dst1, rel_bias1):
    del q_type, pos
    B = utterance_features.shape[0]

    row = lambda v: v.reshape(1, -1)

    grid_spec = pl.GridSpec(
        grid=(B,),
        in_specs=[
            pl.BlockSpec((1, N, EMB), lambda b: (b, 0, 0)),
            pl.BlockSpec((1, N, N), lambda b: (b, 0, 0)),
            pl.BlockSpec((EMB, HID), lambda b: (0, 0)),
            pl.BlockSpec((1, HID), lambda b: (0, 0)),
            pl.BlockSpec((HID, HID), lambda b: (0, 0)),
            pl.BlockSpec((1, HID), lambda b: (0, 0)),
            pl.BlockSpec((1, HID), lambda b: (0, 0)),
            pl.BlockSpec((1, NREL), lambda b: (0, 0)),
            pl.BlockSpec((HID, HID), lambda b: (0, 0)),
            pl.BlockSpec((1, HID), lambda b: (0, 0)),
            pl.BlockSpec((1, HID), lambda b: (0, 0)),
            pl.BlockSpec((1, NREL), lambda b: (0, 0)),
        ],
        out_specs=pl.BlockSpec((1, N, HID + EMB), lambda b: (b, 0, 0)),
    )

    return pl.pallas_call(
        _net_kernel,
        grid_spec=grid_spec,
        out_shape=jax.ShapeDtypeStruct((B, N, HID + EMB), jnp.float32),
    )(utterance_features, semantic_adj,
      W_fc1, row(b_fc1),
      W_gat0, row(a_src0), row(a_dst0), row(rel_bias0),
      W_gat1, row(a_src1), row(a_dst1), row(rel_bias1))
